# baseline (speedup 1.0000x reference)
import functools

import jax
import jax.numpy as jnp
from jax import lax
from jax.experimental import pallas as pl
from jax.experimental.pallas import tpu as pltpu
from jax.experimental.pallas import tpu_sc as plsc

_B, _S, _C = 4096, 50, 1000
_NC = 2
_NS = 16
_NW = _NC * _NS
_BW = _B // _NW
_NG = _BW // 16
_H0 = 504
_H1 = _C - _H0

_mesh = plsc.VectorSubcoreMesh(core_axis_name="c", subcore_axis_name="s")


@functools.partial(
    pl.kernel,
    mesh=_mesh,
    out_type=jax.ShapeDtypeStruct((_S, _C, _B), jnp.int32),
    scratch_types=[
        pltpu.VMEM((3, _BW), jnp.int32),
        pltpu.VMEM((1, _H0, _BW), jnp.int32),
        pltpu.VMEM((1, _H1, _BW), jnp.int32),
        pltpu.SemaphoreType.DMA,
        pltpu.SemaphoreType.DMA,
        pltpu.SemaphoreType.DMA,
    ],
    compiler_params=pltpu.CompilerParams(needs_layout_passes=False),
)
def _sc_onehot(xt_hbm, z_hbm, out_hbm, idx_v, buf0_v, buf1_v, sem0, sem1, semi):
    wid = lax.axis_index("s") * _NC + lax.axis_index("c")
    b0 = wid * _BW
    pltpu.sync_copy(z_hbm.at[:, pl.ds(0, _H0), :], buf0_v)
    pltpu.sync_copy(z_hbm.at[:, pl.ds(_H0, _H1), :], buf1_v)
    pltpu.sync_copy(xt_hbm.at[pl.ds(b0, _BW)], idx_v.at[0])

    lanes = lax.iota(jnp.int32, 16)
    ones = jnp.ones((16,), jnp.int32)
    zeros = jnp.zeros((16,), jnp.int32)
    zero16 = jnp.zeros((16,), jnp.int32)
    halves = ((buf0_v, sem0, 0, _H0), (buf1_v, sem1, _H0, _H1))

    def scat(slot, buf, c0, clen, val):
        def g_body(g, carry):
            bi = g * 16 + lanes
            xv = idx_v[slot, pl.ds(g * 16, 16)]
            m = (xv >= c0) & (xv < c0 + clen)
            row = jnp.minimum(jnp.maximum(xv - c0, 0), clen - 1)
            plsc.store_scatter(buf, [zero16, row, bi], val, mask=m)
            return carry

        lax.fori_loop(0, _NG, g_body, 0)

    def out_dma(buf, sem, s, c0, clen):
        return pltpu.make_async_copy(
            buf,
            out_hbm.at[pl.ds(s, 1), pl.ds(c0, clen), pl.ds(b0, _BW)],
            sem,
        )

    def plane(s, carry):
        slot = lax.rem(s, 3)
        prev = lax.rem(s + 2, 3)
        pltpu.make_async_copy(
            xt_hbm.at[pl.ds(lax.rem(s + 1, _S) * _B + b0, _BW)],
            idx_v.at[lax.rem(s + 1, 3)],
            semi,
        ).start()
        for buf, sem, c0, clen in halves:
            @pl.when(s > 0)
            def _():
                out_dma(buf, sem, s - 1, c0, clen).wait()
                scat(prev, buf, c0, clen, zeros)

            scat(slot, buf, c0, clen, ones)
            out_dma(buf, sem, s, c0, clen).start()
        pltpu.make_async_copy(
            xt_hbm.at[pl.ds(0, _BW)], idx_v.at[0], semi
        ).wait()
        return carry

    lax.fori_loop(0, _S, plane, 0)
    for buf, sem, c0, clen in halves:
        out_dma(buf, sem, _S - 1, c0, clen).wait()


def kernel(x):
    xt = x.T.reshape(_S * _B)
    out_t = _sc_onehot(xt, jnp.zeros((1, _C, _BW), jnp.int32))
    return jnp.transpose(out_t, (2, 0, 1))
